# Initial kernel scaffold; baseline (speedup 1.0000x reference)
import jax
import jax.numpy as jnp
from jax.experimental import pallas as pl


def kernel(x, edge_index, W1, b1, W2, b2):
    raise NotImplementedError("write your pallas kernel here")


# speedup vs baseline: 9.7532x; 9.7532x over previous
import functools

import jax
import jax.numpy as jnp
from jax import lax
from jax.experimental import pallas as pl
from jax.experimental.pallas import tpu as pltpu
from jax.experimental.pallas import tpu_sc as plsc

N = 10000
E = 320000
D = 128

NC = 2
NS = 16
NW = NC * NS
EPW = E // NW
C = 80
NCHUNK = EPW // C
RPT = 624
TAIL = N - NS * RPT
TBASE = NS * RPT

_MESH = plsc.VectorSubcoreMesh(
    core_axis_name="c", subcore_axis_name="s", num_cores=NC, num_subcores=NS
)


@functools.partial(
    pl.kernel,
    out_type=jax.ShapeDtypeStruct((NC, N, D), jnp.float32),
    mesh=_MESH,
    scratch_types=[
        pltpu.VMEM((C,), jnp.int32),
        pltpu.VMEM((C,), jnp.int32),
        pltpu.VMEM((C, D), jnp.float32),
        pltpu.VMEM_SHARED((N, D), jnp.float32),
        pltpu.SemaphoreType.DMA,
    ],
)
def _sc_aggregate(h_hbm, zeros_hbm, src_hbm, dst_hbm, out_hbm,
                  src_v, dst_v, rows_v, acc, sem):
    cid = lax.axis_index("c")
    sid = lax.axis_index("s")
    wid = sid * NC + cid
    rbase = sid * RPT

    @pl.when(cid == 0)
    def _():
        pltpu.sync_copy(h_hbm.at[pl.ds(rbase, RPT)], acc.at[pl.ds(rbase, RPT)])

        @pl.when(sid == NS - 1)
        def _():
            pltpu.sync_copy(h_hbm.at[pl.ds(TBASE, TAIL)], acc.at[pl.ds(TBASE, TAIL)])

    @pl.when(cid != 0)
    def _():
        pltpu.sync_copy(zeros_hbm.at[pl.ds(rbase, RPT)], acc.at[pl.ds(rbase, RPT)])

        @pl.when(sid == NS - 1)
        def _():
            pltpu.sync_copy(zeros_hbm.at[pl.ds(TBASE, TAIL)], acc.at[pl.ds(TBASE, TAIL)])

    plsc.subcore_barrier()

    ebase = wid * EPW

    def chunk(i, carry):
        off = ebase + i * C
        pltpu.sync_copy(src_hbm.at[pl.ds(off, C)], src_v)
        pltpu.sync_copy(dst_hbm.at[pl.ds(off, C)], dst_v)
        pltpu.async_copy(h_hbm.at[src_v], rows_v, sem).wait()
        pltpu.sync_copy(rows_v, acc.at[dst_v], add=True)
        return carry

    lax.fori_loop(0, NCHUNK, chunk, 0)
    plsc.subcore_barrier()
    pltpu.sync_copy(acc.at[pl.ds(rbase, RPT)], out_hbm.at[cid, pl.ds(rbase, RPT)])

    @pl.when(sid == NS - 1)
    def _():
        pltpu.sync_copy(acc.at[pl.ds(TBASE, TAIL)], out_hbm.at[cid, pl.ds(TBASE, TAIL)])


BR = 400
GRID = N // BR


def _dinv_block(degp):
    deg = degp[0, :, :1] + degp[1, :, :1]
    return lax.rsqrt(deg)


def _tc1_body(x_ref, w_ref, degp_ref, out_ref):
    dinv = _dinv_block(degp_ref[...])
    h = jnp.dot(x_ref[...], w_ref[...], preferred_element_type=jnp.float32)
    out_ref[...] = h * dinv


@jax.jit
def _tc1(x, W1, degp):
    return pl.pallas_call(
        _tc1_body,
        grid=(GRID,),
        in_specs=[
            pl.BlockSpec((BR, D), lambda i: (i, 0)),
            pl.BlockSpec((D, D), lambda i: (0, 0)),
            pl.BlockSpec((NC, BR, D), lambda i: (0, i, 0)),
        ],
        out_specs=pl.BlockSpec((BR, D), lambda i: (i, 0)),
        out_shape=jax.ShapeDtypeStruct((N, D), jnp.float32),
    )(x, W1, degp)


def _tc2_body(part_ref, degp_ref, b_ref, w_ref, out_ref):
    dinv = _dinv_block(degp_ref[...])
    s = part_ref[0] + part_ref[1]
    t = jnp.maximum(s * dinv + b_ref[...], 0.0)
    h = jnp.dot(t, w_ref[...], preferred_element_type=jnp.float32)
    out_ref[...] = h * dinv


@jax.jit
def _tc2(part, degp, b1, W2):
    return pl.pallas_call(
        _tc2_body,
        grid=(GRID,),
        in_specs=[
            pl.BlockSpec((NC, BR, D), lambda i: (0, i, 0)),
            pl.BlockSpec((NC, BR, D), lambda i: (0, i, 0)),
            pl.BlockSpec((D,), lambda i: (0,)),
            pl.BlockSpec((D, D), lambda i: (0, 0)),
        ],
        out_specs=pl.BlockSpec((BR, D), lambda i: (i, 0)),
        out_shape=jax.ShapeDtypeStruct((N, D), jnp.float32),
    )(part, degp, b1, W2)


def _tc3_body(part_ref, degp_ref, b_ref, out_ref):
    dinv = _dinv_block(degp_ref[...])
    s = part_ref[0] + part_ref[1]
    out_ref[...] = s * dinv + b_ref[...]


@jax.jit
def _tc3(part, degp, b2):
    return pl.pallas_call(
        _tc3_body,
        grid=(GRID,),
        in_specs=[
            pl.BlockSpec((NC, BR, D), lambda i: (0, i, 0)),
            pl.BlockSpec((NC, BR, D), lambda i: (0, i, 0)),
            pl.BlockSpec((D,), lambda i: (0,)),
        ],
        out_specs=pl.BlockSpec((BR, D), lambda i: (i, 0)),
        out_shape=jax.ShapeDtypeStruct((N, D), jnp.float32),
    )(part, degp, b2)


def kernel(x, edge_index, W1, b1, W2, b2):
    src = edge_index[0]
    dst = edge_index[1]
    zeros_nd = jnp.zeros((N, D), jnp.float32)
    ones_nd = jnp.ones((N, D), jnp.float32)

    degp = _sc_aggregate(ones_nd, zeros_nd, src, dst)
    h1 = _tc1(x, W1, degp)
    p1 = _sc_aggregate(h1, zeros_nd, src, dst)
    h2 = _tc2(p1, degp, b1, W2)
    p2 = _sc_aggregate(h2, zeros_nd, src, dst)
    return _tc3(p2, degp, b2)


# speedup vs baseline: 10.2642x; 1.0524x over previous
import functools

import jax
import jax.numpy as jnp
from jax import lax
from jax.experimental import pallas as pl
from jax.experimental.pallas import tpu as pltpu
from jax.experimental.pallas import tpu_sc as plsc

N = 10000
E = 320000
D = 128

NC = 2
NS = 16
NW = NC * NS
C = 128
EROWS = 2560
EPAD = EROWS * C - E
RPW = EROWS // NW
RPP = RPW // 2
NP = N + 8
RPT = 624
TAIL = N - NS * RPT
TBASE = NS * RPT

_MESH = plsc.VectorSubcoreMesh(
    core_axis_name="c", subcore_axis_name="s", num_cores=NC, num_subcores=NS
)


def _seed_acc(cid, sid, h_hbm, zeros_hbm, acc):
    rbase = sid * RPT

    @pl.when(cid == 0)
    def _():
        pltpu.sync_copy(h_hbm.at[pl.ds(rbase, RPT)], acc.at[pl.ds(rbase, RPT)])

        @pl.when(sid == NS - 1)
        def _():
            pltpu.sync_copy(h_hbm.at[pl.ds(TBASE, TAIL)], acc.at[pl.ds(TBASE, TAIL)])

    @pl.when(cid != 0)
    def _():
        pltpu.sync_copy(zeros_hbm.at[pl.ds(rbase, RPT)], acc.at[pl.ds(rbase, RPT)])

        @pl.when(sid == NS - 1)
        def _():
            pltpu.sync_copy(zeros_hbm.at[pl.ds(TBASE, TAIL)], acc.at[pl.ds(TBASE, TAIL)])


def _copy_out(cid, sid, acc, out_hbm):
    rbase = sid * RPT
    pltpu.sync_copy(acc.at[pl.ds(rbase, RPT)], out_hbm.at[cid, pl.ds(rbase, RPT)])

    @pl.when(sid == NS - 1)
    def _():
        pltpu.sync_copy(acc.at[pl.ds(TBASE, TAIL)], out_hbm.at[cid, pl.ds(TBASE, TAIL)])


@functools.partial(
    pl.kernel,
    out_type=jax.ShapeDtypeStruct((NC, N, D), jnp.float32),
    mesh=_MESH,
    scratch_types=[
        pltpu.VMEM((RPP, C), jnp.int32),
        pltpu.VMEM((RPP, C), jnp.int32),
        pltpu.VMEM((C, D), jnp.float32),
        pltpu.VMEM((C, D), jnp.float32),
        pltpu.VMEM_SHARED((NP, D), jnp.float32),
        pltpu.SemaphoreType.DMA,
        pltpu.SemaphoreType.DMA,
    ],
)
def _sc_aggregate(h_hbm, zeros_hbm, src_hbm, dst_hbm, out_hbm,
                  src_t, dst_t, rows0, rows1, acc, sem0, sem1):
    cid = lax.axis_index("c")
    sid = lax.axis_index("s")
    wid = sid * NC + cid

    _seed_acc(cid, sid, h_hbm, zeros_hbm, acc)
    plsc.subcore_barrier()

    for half in range(RPW // RPP):
        rb = wid * RPW + half * RPP
        pltpu.sync_copy(src_hbm.at[pl.ds(rb, RPP)], src_t)
        pltpu.sync_copy(dst_hbm.at[pl.ds(rb, RPP)], dst_t)

        pltpu.async_copy(h_hbm.at[src_t.at[0]], rows0, sem0)

        def pair(p, carry):
            i0 = 2 * p
            pltpu.async_copy(h_hbm.at[src_t.at[i0 + 1]], rows1, sem1)
            pltpu.make_async_copy(h_hbm.at[src_t.at[i0]], rows0, sem0).wait()
            pltpu.sync_copy(rows0, acc.at[dst_t.at[i0]], add=True)

            @pl.when(i0 + 2 < RPP)
            def _():
                pltpu.async_copy(h_hbm.at[src_t.at[i0 + 2]], rows0, sem0)

            pltpu.make_async_copy(h_hbm.at[src_t.at[i0 + 1]], rows1, sem1).wait()
            pltpu.sync_copy(rows1, acc.at[dst_t.at[i0 + 1]], add=True)
            return carry

        lax.fori_loop(0, RPP // 2, pair, 0)

    plsc.subcore_barrier()
    _copy_out(cid, sid, acc, out_hbm)


@functools.partial(
    pl.kernel,
    out_type=jax.ShapeDtypeStruct((NC, N, D), jnp.float32),
    mesh=_MESH,
    scratch_types=[
        pltpu.VMEM((RPW, C), jnp.int32),
        pltpu.VMEM((C, D), jnp.float32),
        pltpu.VMEM_SHARED((NP, D), jnp.float32),
    ],
)
def _sc_degree(dst_hbm, zeros_hbm, ones_hbm, out_hbm, dst_t, ones_t, acc):
    cid = lax.axis_index("c")
    sid = lax.axis_index("s")
    wid = sid * NC + cid

    _seed_acc(cid, sid, zeros_hbm, zeros_hbm, acc)
    pltpu.sync_copy(dst_hbm.at[pl.ds(wid * RPW, RPW)], dst_t)
    pltpu.sync_copy(ones_hbm, ones_t)
    plsc.subcore_barrier()

    def chunk(i, carry):
        pltpu.sync_copy(ones_t, acc.at[dst_t.at[i]], add=True)
        return carry

    lax.fori_loop(0, RPW, chunk, 0)
    plsc.subcore_barrier()
    _copy_out(cid, sid, acc, out_hbm)


BR = 400
GRID = N // BR


def _dinv_block(degp):
    deg = degp[0, :, :1] + degp[1, :, :1] + 1.0
    return lax.rsqrt(deg)


def _tc1_body(x_ref, w_ref, degp_ref, out_ref):
    dinv = _dinv_block(degp_ref[...])
    h = jnp.dot(x_ref[...], w_ref[...], preferred_element_type=jnp.float32)
    out_ref[...] = h * dinv


@jax.jit
def _tc1(x, W1, degp):
    return pl.pallas_call(
        _tc1_body,
        grid=(GRID,),
        in_specs=[
            pl.BlockSpec((BR, D), lambda i: (i, 0)),
            pl.BlockSpec((D, D), lambda i: (0, 0)),
            pl.BlockSpec((NC, BR, D), lambda i: (0, i, 0)),
        ],
        out_specs=pl.BlockSpec((BR, D), lambda i: (i, 0)),
        out_shape=jax.ShapeDtypeStruct((N, D), jnp.float32),
    )(x, W1, degp)


def _tc2_body(part_ref, degp_ref, b_ref, w_ref, out_ref):
    dinv = _dinv_block(degp_ref[...])
    s = part_ref[0] + part_ref[1]
    t = jnp.maximum(s * dinv + b_ref[...], 0.0)
    h = jnp.dot(t, w_ref[...], preferred_element_type=jnp.float32)
    out_ref[...] = h * dinv


@jax.jit
def _tc2(part, degp, b1, W2):
    return pl.pallas_call(
        _tc2_body,
        grid=(GRID,),
        in_specs=[
            pl.BlockSpec((NC, BR, D), lambda i: (0, i, 0)),
            pl.BlockSpec((NC, BR, D), lambda i: (0, i, 0)),
            pl.BlockSpec((D,), lambda i: (0,)),
            pl.BlockSpec((D, D), lambda i: (0, 0)),
        ],
        out_specs=pl.BlockSpec((BR, D), lambda i: (i, 0)),
        out_shape=jax.ShapeDtypeStruct((N, D), jnp.float32),
    )(part, degp, b1, W2)


def _tc3_body(part_ref, degp_ref, b_ref, out_ref):
    dinv = _dinv_block(degp_ref[...])
    s = part_ref[0] + part_ref[1]
    out_ref[...] = s * dinv + b_ref[...]


@jax.jit
def _tc3(part, degp, b2):
    return pl.pallas_call(
        _tc3_body,
        grid=(GRID,),
        in_specs=[
            pl.BlockSpec((NC, BR, D), lambda i: (0, i, 0)),
            pl.BlockSpec((NC, BR, D), lambda i: (0, i, 0)),
            pl.BlockSpec((D,), lambda i: (0,)),
        ],
        out_specs=pl.BlockSpec((BR, D), lambda i: (i, 0)),
        out_shape=jax.ShapeDtypeStruct((N, D), jnp.float32),
    )(part, degp, b2)


def kernel(x, edge_index, W1, b1, W2, b2):
    src = edge_index[0]
    dst = edge_index[1]
    pad_src = jnp.zeros((EPAD,), jnp.int32)
    pad_dst = jnp.full((EPAD,), N, jnp.int32)
    src2d = jnp.concatenate([src, pad_src]).reshape(EROWS, C)
    dst2d = jnp.concatenate([dst, pad_dst]).reshape(EROWS, C)

    zeros_nd = jnp.zeros((N, D), jnp.float32)
    ones_cd = jnp.ones((C, D), jnp.float32)

    degp = _sc_degree(dst2d, zeros_nd, ones_cd)
    h1 = _tc1(x, W1, degp)
    p1 = _sc_aggregate(h1, zeros_nd, src2d, dst2d)
    h2 = _tc2(p1, degp, b1, W2)
    p2 = _sc_aggregate(h2, zeros_nd, src2d, dst2d)
    return _tc3(p2, degp, b2)


# speedup vs baseline: 10.4858x; 1.0216x over previous
import functools

import jax
import jax.numpy as jnp
from jax import lax
from jax.experimental import pallas as pl
from jax.experimental.pallas import tpu as pltpu
from jax.experimental.pallas import tpu_sc as plsc

N = 10000
E = 320000
D = 128

NC = 2
NS = 16
NW = NC * NS
C = 128
EROWS = 2560
EPAD = EROWS * C - E
RPW = EROWS // NW
RPP = RPW // 2
NP = N + 8
RPT = 624
TAIL = N - NS * RPT
TBASE = NS * RPT

_MESH = plsc.VectorSubcoreMesh(
    core_axis_name="c", subcore_axis_name="s", num_cores=NC, num_subcores=NS
)


def _seed_acc(cid, sid, zeros_hbm, acc):
    del cid
    rbase = sid * RPT
    pltpu.sync_copy(zeros_hbm.at[pl.ds(rbase, RPT)], acc.at[pl.ds(rbase, RPT)])

    @pl.when(sid == NS - 1)
    def _():
        pltpu.sync_copy(zeros_hbm.at[pl.ds(TBASE, TAIL)], acc.at[pl.ds(TBASE, TAIL)])


def _copy_out(cid, sid, acc, out_hbm):
    rbase = sid * RPT
    pltpu.sync_copy(acc.at[pl.ds(rbase, RPT)], out_hbm.at[cid, pl.ds(rbase, RPT)])

    @pl.when(sid == NS - 1)
    def _():
        pltpu.sync_copy(acc.at[pl.ds(TBASE, TAIL)], out_hbm.at[cid, pl.ds(TBASE, TAIL)])


@functools.partial(
    pl.kernel,
    out_type=jax.ShapeDtypeStruct((NC, N, D), jnp.float32),
    mesh=_MESH,
    scratch_types=[
        pltpu.VMEM((RPP, C), jnp.int32),
        pltpu.VMEM((RPP, C), jnp.int32),
        pltpu.VMEM((C, D), jnp.float32),
        pltpu.VMEM((C, D), jnp.float32),
        pltpu.VMEM_SHARED((NP, D), jnp.float32),
        pltpu.SemaphoreType.DMA,
        pltpu.SemaphoreType.DMA,
    ],
)
def _sc_aggregate(h_hbm, zeros_hbm, src_hbm, dst_hbm, out_hbm,
                  src_t, dst_t, rows0, rows1, acc, sem0, sem1):
    cid = lax.axis_index("c")
    sid = lax.axis_index("s")
    wid = sid * NC + cid

    _seed_acc(cid, sid, zeros_hbm, acc)
    plsc.subcore_barrier()

    for half in range(RPW // RPP):
        rb = wid * RPW + half * RPP
        pltpu.sync_copy(src_hbm.at[pl.ds(rb, RPP)], src_t)
        pltpu.sync_copy(dst_hbm.at[pl.ds(rb, RPP)], dst_t)

        pltpu.async_copy(h_hbm.at[src_t.at[0]], rows0, sem0)

        def pair(p, carry):
            i0 = 2 * p
            pltpu.async_copy(h_hbm.at[src_t.at[i0 + 1]], rows1, sem1)
            pltpu.make_async_copy(h_hbm.at[src_t.at[i0]], rows0, sem0).wait()
            pltpu.sync_copy(rows0, acc.at[dst_t.at[i0]], add=True)

            @pl.when(i0 + 2 < RPP)
            def _():
                pltpu.async_copy(h_hbm.at[src_t.at[i0 + 2]], rows0, sem0)

            pltpu.make_async_copy(h_hbm.at[src_t.at[i0 + 1]], rows1, sem1).wait()
            pltpu.sync_copy(rows1, acc.at[dst_t.at[i0 + 1]], add=True)
            return carry

        lax.fori_loop(0, RPP // 2, pair, 0)

    plsc.subcore_barrier()
    _copy_out(cid, sid, acc, out_hbm)


@functools.partial(
    pl.kernel,
    out_type=jax.ShapeDtypeStruct((NC, N, D), jnp.float32),
    mesh=_MESH,
    scratch_types=[
        pltpu.VMEM((RPW, C), jnp.int32),
        pltpu.VMEM((C, D), jnp.float32),
        pltpu.VMEM_SHARED((NP, D), jnp.float32),
    ],
)
def _sc_degree(dst_hbm, zeros_hbm, ones_hbm, out_hbm, dst_t, ones_t, acc):
    cid = lax.axis_index("c")
    sid = lax.axis_index("s")
    wid = sid * NC + cid

    _seed_acc(cid, sid, zeros_hbm, acc)
    pltpu.sync_copy(dst_hbm.at[pl.ds(wid * RPW, RPW)], dst_t)
    pltpu.sync_copy(ones_hbm, ones_t)
    plsc.subcore_barrier()

    def chunk(i, carry):
        pltpu.sync_copy(ones_t, acc.at[dst_t.at[i]], add=True)
        return carry

    lax.fori_loop(0, RPW, chunk, 0)
    plsc.subcore_barrier()
    _copy_out(cid, sid, acc, out_hbm)


BR = 400
GRID = N // BR


def _dinv_block(degp):
    deg = degp[0, :, :1] + degp[1, :, :1] + 1.0
    return lax.rsqrt(deg)


def _tc1_body(x_ref, w_ref, degp_ref, out_ref):
    dinv = _dinv_block(degp_ref[...])
    h = jnp.dot(x_ref[...], w_ref[...], preferred_element_type=jnp.float32)
    out_ref[...] = h * dinv


@jax.jit
def _tc1(x, W1, degp):
    return pl.pallas_call(
        _tc1_body,
        grid=(GRID,),
        in_specs=[
            pl.BlockSpec((BR, D), lambda i: (i, 0)),
            pl.BlockSpec((D, D), lambda i: (0, 0)),
            pl.BlockSpec((NC, BR, D), lambda i: (0, i, 0)),
        ],
        out_specs=pl.BlockSpec((BR, D), lambda i: (i, 0)),
        out_shape=jax.ShapeDtypeStruct((N, D), jnp.float32),
    )(x, W1, degp)


def _tc2_body(part_ref, h_ref, degp_ref, b_ref, w_ref, out_ref):
    dinv = _dinv_block(degp_ref[...])
    s = part_ref[0] + part_ref[1] + h_ref[...]
    t = jnp.maximum(s * dinv + b_ref[...], 0.0)
    h = jnp.dot(t, w_ref[...], preferred_element_type=jnp.float32)
    out_ref[...] = h * dinv


@jax.jit
def _tc2(part, h1, degp, b1, W2):
    return pl.pallas_call(
        _tc2_body,
        grid=(GRID,),
        in_specs=[
            pl.BlockSpec((NC, BR, D), lambda i: (0, i, 0)),
            pl.BlockSpec((BR, D), lambda i: (i, 0)),
            pl.BlockSpec((NC, BR, D), lambda i: (0, i, 0)),
            pl.BlockSpec((D,), lambda i: (0,)),
            pl.BlockSpec((D, D), lambda i: (0, 0)),
        ],
        out_specs=pl.BlockSpec((BR, D), lambda i: (i, 0)),
        out_shape=jax.ShapeDtypeStruct((N, D), jnp.float32),
    )(part, h1, degp, b1, W2)


def _tc3_body(part_ref, h_ref, degp_ref, b_ref, out_ref):
    dinv = _dinv_block(degp_ref[...])
    s = part_ref[0] + part_ref[1] + h_ref[...]
    out_ref[...] = s * dinv + b_ref[...]


@jax.jit
def _tc3(part, h2, degp, b2):
    return pl.pallas_call(
        _tc3_body,
        grid=(GRID,),
        in_specs=[
            pl.BlockSpec((NC, BR, D), lambda i: (0, i, 0)),
            pl.BlockSpec((BR, D), lambda i: (i, 0)),
            pl.BlockSpec((NC, BR, D), lambda i: (0, i, 0)),
            pl.BlockSpec((D,), lambda i: (0,)),
        ],
        out_specs=pl.BlockSpec((BR, D), lambda i: (i, 0)),
        out_shape=jax.ShapeDtypeStruct((N, D), jnp.float32),
    )(part, h2, degp, b2)


def kernel(x, edge_index, W1, b1, W2, b2):
    src = edge_index[0]
    dst = edge_index[1]
    pad_src = jnp.zeros((EPAD,), jnp.int32)
    pad_dst = jnp.full((EPAD,), N, jnp.int32)
    src2d = jnp.concatenate([src, pad_src]).reshape(EROWS, C)
    dst2d = jnp.concatenate([dst, pad_dst]).reshape(EROWS, C)

    zeros_nd = jnp.zeros((N, D), jnp.float32)
    ones_cd = jnp.ones((C, D), jnp.float32)

    degp = _sc_degree(dst2d, zeros_nd, ones_cd)
    h1 = _tc1(x, W1, degp)
    p1 = _sc_aggregate(h1, zeros_nd, src2d, dst2d)
    h2 = _tc2(p1, h1, degp, b1, W2)
    p2 = _sc_aggregate(h2, zeros_nd, src2d, dst2d)
    return _tc3(p2, h2, degp, b2)
